# baseline (speedup 1.0000x reference)
import jax
import jax.numpy as jnp
from jax.experimental import pallas as pl
from jax.experimental.pallas import tpu as pltpu


def _gc_kernel(v_ref, w_ref, b_ref, ad_ref, aa_ref, o_ref, s_ref):
    @pl.when(pl.program_id(0) == 0)
    def _():
        s_ref[...] = jnp.dot(v_ref[...].astype(jnp.bfloat16),
                             w_ref[...].astype(jnp.bfloat16),
                             preferred_element_type=jnp.float32
                             ).astype(jnp.bfloat16)

    a = (ad_ref[...] + aa_ref[...]).astype(jnp.bfloat16)
    o_ref[...] = (jnp.dot(a, s_ref[...], preferred_element_type=jnp.float32)
                  + b_ref[...])


def kernel(vertex, adj_distance, adj_angle, weights, bias):
    n, in_f = vertex.shape
    out_f = weights.shape[1]
    bias2 = bias.reshape(1, out_f)

    tm = 200
    grid = (n // tm,)

    return pl.pallas_call(
        _gc_kernel,
        grid=grid,
        in_specs=[
            pl.BlockSpec((n, in_f), lambda m: (0, 0)),
            pl.BlockSpec((in_f, out_f), lambda m: (0, 0)),
            pl.BlockSpec((1, out_f), lambda m: (0, 0)),
            pl.BlockSpec((tm, n), lambda m: (m, 0)),
            pl.BlockSpec((tm, n), lambda m: (m, 0)),
        ],
        out_specs=pl.BlockSpec((tm, out_f), lambda m: (m, 0)),
        out_shape=jax.ShapeDtypeStruct((n, out_f), jnp.float32),
        scratch_shapes=[pltpu.VMEM((n, out_f), jnp.bfloat16)],
        compiler_params=pltpu.CompilerParams(
            dimension_semantics=("arbitrary",),
        ),
    )(vertex, weights, bias2, adj_distance, adj_angle)
